# Initial kernel scaffold; baseline (speedup 1.0000x reference)
import functools

import jax
import jax.numpy as jnp
from jax import lax
from jax.experimental import pallas as pl
from jax.experimental.pallas import tpu as pltpu
from jax.experimental.pallas import tpu_sc as plsc

N = 10000
E = 320000
D = 128
B = 64
C = 10

_NC = 2
_NS = 16
_NW = _NC * _NS
_EPW = E // _NW
_K = 80
_NBLK = _EPW // _K
_RPT = N // _NS


_EBLK = 4000


def _edge_body(ea_ref, w0_ref, c0_ref, w1_ref, c1_ref, msg0_ref, e1_ref):
    a = ea_ref[...]
    m0 = jnp.dot(a, w0_ref[...], preferred_element_type=jnp.float32)
    msg0_ref[...] = jnp.maximum(m0 + c0_ref[...], 0.0)
    e1 = jnp.dot(a, w1_ref[...], preferred_element_type=jnp.float32)
    e1_ref[...] = e1 + c1_ref[...]


def _edge_embed(ea8, w0, c0, w1, c1):
    return pl.pallas_call(
        _edge_body,
        grid=(E // _EBLK,),
        in_specs=[
            pl.BlockSpec((_EBLK, 8), lambda i: (i, 0)),
            pl.BlockSpec((8, D), lambda i: (0, 0)),
            pl.BlockSpec((1, D), lambda i: (0, 0)),
            pl.BlockSpec((8, D), lambda i: (0, 0)),
            pl.BlockSpec((1, D), lambda i: (0, 0)),
        ],
        out_specs=[
            pl.BlockSpec((_EBLK, D), lambda i: (i, 0)),
            pl.BlockSpec((_EBLK, D), lambda i: (i, 0)),
        ],
        out_shape=[
            jax.ShapeDtypeStruct((E, D), jnp.float32),
            jax.ShapeDtypeStruct((E, D), jnp.float32),
        ],
    )(ea8, w0, c0, w1, c1)


def _sc_scatter_add(vals, idx, zeros_nd):
    mesh = plsc.VectorSubcoreMesh(core_axis_name="c", subcore_axis_name="s")

    @functools.partial(
        pl.kernel,
        mesh=mesh,
        out_type=jax.ShapeDtypeStruct((_NC, N, D), jnp.float32),
        scratch_types=[
            pltpu.VMEM((_K, D), jnp.float32),
            pltpu.VMEM((_K,), jnp.int32),
            pltpu.VMEM_SHARED((N, D), jnp.float32),
        ],
    )
    def k(vals_hbm, idx_hbm, zeros_hbm, out_hbm, vbuf, ibuf, acc):
        c = lax.axis_index("c")
        s = lax.axis_index("s")
        wid = s * _NC + c
        r0 = s * _RPT
        pltpu.sync_copy(zeros_hbm.at[pl.ds(r0, _RPT)], acc.at[pl.ds(r0, _RPT)])
        plsc.subcore_barrier()
        e0 = wid * _EPW

        def body(i, carry):
            e = e0 + i * _K
            pltpu.sync_copy(vals_hbm.at[pl.ds(e, _K)], vbuf)
            pltpu.sync_copy(idx_hbm.at[pl.ds(e, _K)], ibuf)
            pltpu.sync_copy(vbuf, acc.at[ibuf], add=True)
            return carry

        lax.fori_loop(0, _NBLK, body, 0)
        plsc.subcore_barrier()
        pltpu.sync_copy(acc.at[pl.ds(r0, _RPT)],
                        out_hbm.at[c].at[pl.ds(r0, _RPT)])

    return k(vals, idx, zeros_nd)


def _sc_gather_msg_scatter(h, eemb, src, dst, zeros_nd):
    mesh = plsc.VectorSubcoreMesh(core_axis_name="c", subcore_axis_name="s")

    @functools.partial(
        pl.kernel,
        mesh=mesh,
        out_type=jax.ShapeDtypeStruct((_NC, N, D), jnp.float32),
        scratch_types=[
            pltpu.VMEM((_K, D), jnp.float32),
            pltpu.VMEM((_K, D), jnp.float32),
            pltpu.VMEM((_K,), jnp.int32),
            pltpu.VMEM((_K,), jnp.int32),
            pltpu.VMEM_SHARED((N, D), jnp.float32),
            pltpu.SemaphoreType.DMA,
        ],
    )
    def k(h_hbm, eemb_hbm, src_hbm, dst_hbm, zeros_hbm, out_hbm,
          gbuf, ebuf, isrc, idst, acc, sem):
        c = lax.axis_index("c")
        s = lax.axis_index("s")
        wid = s * _NC + c
        r0 = s * _RPT
        pltpu.sync_copy(zeros_hbm.at[pl.ds(r0, _RPT)], acc.at[pl.ds(r0, _RPT)])
        plsc.subcore_barrier()
        e0 = wid * _EPW

        def body(i, carry):
            e = e0 + i * _K
            pltpu.sync_copy(src_hbm.at[pl.ds(e, _K)], isrc)
            pltpu.sync_copy(dst_hbm.at[pl.ds(e, _K)], idst)
            pltpu.sync_copy(eemb_hbm.at[pl.ds(e, _K)], ebuf)
            pltpu.async_copy(h_hbm.at[isrc], gbuf, sem).wait()

            def row(r, rc):
                for c8 in range(D // 16):
                    sl = pl.ds(c8 * 16, 16)
                    ebuf[r, sl] = jnp.maximum(ebuf[r, sl] + gbuf[r, sl], 0.0)
                return rc

            lax.fori_loop(0, _K, row, 0)
            pltpu.sync_copy(ebuf, acc.at[idst], add=True)
            return carry

        lax.fori_loop(0, _NBLK, body, 0)
        plsc.subcore_barrier()
        pltpu.sync_copy(acc.at[pl.ds(r0, _RPT)],
                        out_hbm.at[c].at[pl.ds(r0, _RPT)])

    return k(h, eemb, src, dst, zeros_nd)


def _bn_tc(t, g, b):
    m = jnp.mean(t, axis=0, keepdims=True)
    v = jnp.mean(t * t, axis=0, keepdims=True) - m * m
    return (t - m) * jax.lax.rsqrt(v + 1e-5) * g + b


def _mlp_body(final_relu, agg_ref, h_ref, sc_ref, w1_ref, b1_ref, g1_ref,
              t1_ref, w2_ref, b2_ref, g2_ref, t2_ref, out_ref):
    pre = (agg_ref[0] + agg_ref[1]) + sc_ref[0, 0] * h_ref[...]
    t = jnp.dot(pre, w1_ref[...], preferred_element_type=jnp.float32)
    t = jnp.maximum(_bn_tc(t + b1_ref[...], g1_ref[...], t1_ref[...]), 0.0)
    t = jnp.dot(t, w2_ref[...], preferred_element_type=jnp.float32)
    t = _bn_tc(t + b2_ref[...], g2_ref[...], t2_ref[...])
    if final_relu:
        t = jnp.maximum(t, 0.0)
    out_ref[...] = t


def _mlp(agg, h, scale, w1, b1, g1, t1, w2, b2, g2, t2, final_relu):
    nsmem = pl.BlockSpec(memory_space=pltpu.SMEM)
    args = (agg, h, scale, w1, b1, g1, t1, w2, b2, g2, t2)
    in_specs = [nsmem if a is scale else pl.BlockSpec(a.shape, None)
                for a in args]
    return pl.pallas_call(
        functools.partial(_mlp_body, final_relu),
        in_specs=in_specs,
        out_shape=jax.ShapeDtypeStruct((N, D), jnp.float32),
    )(*args)


def _mlp_pool_body(agg_ref, h_ref, sc_ref, w1_ref, b1_ref, g1_ref, t1_ref,
                   w2_ref, b2_ref, g2_ref, t2_ref, batch_ref, wp1_ref,
                   bp1_ref, wp_ref, bp_ref, out_ref, g_ref, hn_ref):
    pre = (agg_ref[0] + agg_ref[1]) + sc_ref[0, 0] * h_ref[...]
    t = jnp.dot(pre, w1_ref[...], preferred_element_type=jnp.float32)
    t = jnp.maximum(_bn_tc(t + b1_ref[...], g1_ref[...], t1_ref[...]), 0.0)
    t = jnp.dot(t, w2_ref[...], preferred_element_type=jnp.float32)
    hn = _bn_tc(t + b2_ref[...], g2_ref[...], t2_ref[...])
    hn_ref[...] = hn
    iota = lax.broadcasted_iota(jnp.int32, (N, B), 1)
    oh = (batch_ref[...] == iota).astype(jnp.float32)
    dn = (((0,), (0,)), ((), ()))
    sums = lax.dot_general(oh, hn, dn, preferred_element_type=jnp.float32)
    cnt = lax.dot_general(oh, jnp.ones((N, 1), jnp.float32), dn,
                          preferred_element_type=jnp.float32)
    hg = sums / jnp.maximum(cnt, 1.0)
    gg = jnp.maximum(hg, 0.0)
    gg = jnp.dot(gg, wp1_ref[...], preferred_element_type=jnp.float32)
    gg = jnp.maximum(gg + bp1_ref[...], 0.0)
    g_ref[...] = gg
    out = jnp.dot(gg, wp_ref[...], preferred_element_type=jnp.float32)
    out_ref[...] = out + bp_ref[...]


def _mlp_pool(agg, h, scale, w1, b1, g1, t1, w2, b2, g2, t2,
              batch2d, wp1, bp1, wp, bp):
    nsmem = pl.BlockSpec(memory_space=pltpu.SMEM)
    args = (agg, h, scale, w1, b1, g1, t1, w2, b2, g2, t2,
            batch2d, wp1, bp1, wp, bp)
    in_specs = [nsmem if a is scale else pl.BlockSpec(a.shape, None)
                for a in args]
    return pl.pallas_call(
        _mlp_pool_body,
        in_specs=in_specs,
        out_shape=[
            jax.ShapeDtypeStruct((B, C), jnp.float32),
            jax.ShapeDtypeStruct((B, D), jnp.float32),
            jax.ShapeDtypeStruct((N, D), jnp.float32),
        ],
    )(*args)


def kernel(x, edge_index, edge_attr, batch, node_table, We, be, eps,
           W1, b1, g1, bt1, W2, b2, g2, bt2, Wp1, bp1, Wp, bp):
    f32 = jnp.float32
    src = edge_index[0].astype(jnp.int32)
    dst = edge_index[1].astype(jnp.int32)
    ea8 = jnp.concatenate([edge_attr, jnp.zeros((E, 1), f32)], axis=1)
    w0 = jnp.concatenate([We[0], jnp.zeros((1, D), f32)], axis=0)
    w1e = jnp.concatenate([We[1], jnp.zeros((1, D), f32)], axis=0)
    c0 = node_table + be[0][None, :]
    c1 = be[1][None, :]
    zeros_nd = jnp.zeros((N, D), f32)
    batch2d = batch.astype(jnp.int32)[:, None]

    msg0, eemb1 = _edge_embed(ea8, w0, c0, w1e, c1)
    agg0 = _sc_scatter_add(msg0, dst, zeros_nd)
    sc0 = (1.0 + eps[0]).astype(f32).reshape(1, 1)
    h1 = _mlp(agg0, node_table, sc0, W1[0], b1[0][None], g1[0][None],
              bt1[0][None], W2[0], b2[0][None], g2[0][None], bt2[0][None],
              final_relu=True)
    agg1 = _sc_gather_msg_scatter(h1, eemb1, src, dst, zeros_nd)
    sc1 = (1.0 + eps[1]).astype(f32).reshape(1, 1)
    out, g, h_node = _mlp_pool(
        agg1, h1, sc1, W1[1], b1[1][None], g1[1][None], bt1[1][None],
        W2[1], b2[1][None], g2[1][None], bt2[1][None],
        batch2d, Wp1, bp1[None], Wp, bp[None])
    return out, g, h_node


# speedup vs baseline: 3.1099x; 3.1099x over previous
import functools

import jax
import jax.numpy as jnp
from jax import lax
from jax.experimental import pallas as pl
from jax.experimental.pallas import tpu as pltpu
from jax.experimental.pallas import tpu_sc as plsc

N = 10000
E = 320000
D = 128
B = 64
C = 10

_NC = 2
_NS = 16
_NW = _NC * _NS
_EPW = E // _NW
_K = 80
_NBLK = _EPW // _K
_RPT = 624
_TAIL0 = _RPT * _NS
_TAIL = N - _TAIL0


def _rows_copy(src_ref, dst_ref, s):
    r0 = s * _RPT
    pltpu.sync_copy(src_ref.at[pl.ds(r0, _RPT)], dst_ref.at[pl.ds(r0, _RPT)])

    @pl.when(s == _NS - 1)
    def _():
        pltpu.sync_copy(src_ref.at[pl.ds(_TAIL0, _TAIL)],
                        dst_ref.at[pl.ds(_TAIL0, _TAIL)])


_EBLK = 4000


def _edge_body(ea_ref, w0_ref, c0_ref, w1_ref, c1_ref, msg0_ref, e1_ref):
    a = ea_ref[...]
    m0 = jnp.dot(a, w0_ref[...], preferred_element_type=jnp.float32)
    msg0_ref[...] = jnp.maximum(m0 + c0_ref[...], 0.0)
    e1 = jnp.dot(a, w1_ref[...], preferred_element_type=jnp.float32)
    e1_ref[...] = e1 + c1_ref[...]


def _edge_embed(ea8, w0, c0, w1, c1):
    return pl.pallas_call(
        _edge_body,
        grid=(E // _EBLK,),
        in_specs=[
            pl.BlockSpec((_EBLK, 8), lambda i: (i, 0)),
            pl.BlockSpec((8, D), lambda i: (0, 0)),
            pl.BlockSpec((1, D), lambda i: (0, 0)),
            pl.BlockSpec((8, D), lambda i: (0, 0)),
            pl.BlockSpec((1, D), lambda i: (0, 0)),
        ],
        out_specs=[
            pl.BlockSpec((_EBLK, D), lambda i: (i, 0)),
            pl.BlockSpec((_EBLK, D), lambda i: (i, 0)),
        ],
        out_shape=[
            jax.ShapeDtypeStruct((E, D), jnp.float32),
            jax.ShapeDtypeStruct((E, D), jnp.float32),
        ],
    )(ea8, w0, c0, w1, c1)


def _sc_scatter_add(vals, idx, zeros_nd):
    mesh = plsc.VectorSubcoreMesh(core_axis_name="c", subcore_axis_name="s")

    @functools.partial(
        pl.kernel,
        mesh=mesh,
        out_type=jax.ShapeDtypeStruct((_NC, N, D), jnp.float32),
        scratch_types=[
            pltpu.VMEM((_K, D), jnp.float32),
            pltpu.VMEM((_K,), jnp.int32),
            pltpu.VMEM_SHARED((N, D), jnp.float32),
        ],
    )
    def k(vals_hbm, idx_hbm, zeros_hbm, out_hbm, vbuf, ibuf, acc):
        c = lax.axis_index("c")
        s = lax.axis_index("s")
        wid = s * _NC + c
        _rows_copy(zeros_hbm, acc, s)
        plsc.subcore_barrier()
        e0 = wid * _EPW

        def body(i, carry):
            e = e0 + i * _K
            pltpu.sync_copy(vals_hbm.at[pl.ds(e, _K)], vbuf)
            pltpu.sync_copy(idx_hbm.at[pl.ds(e, _K)], ibuf)
            pltpu.sync_copy(vbuf, acc.at[ibuf], add=True)
            return carry

        lax.fori_loop(0, _NBLK, body, 0)
        plsc.subcore_barrier()
        _rows_copy(acc, out_hbm.at[c], s)

    return k(vals, idx, zeros_nd)


def _sc_gather_msg_scatter(h, eemb, src, dst, zeros_nd):
    mesh = plsc.VectorSubcoreMesh(core_axis_name="c", subcore_axis_name="s")

    @functools.partial(
        pl.kernel,
        mesh=mesh,
        out_type=jax.ShapeDtypeStruct((_NC, N, D), jnp.float32),
        scratch_types=[
            pltpu.VMEM((_K, D), jnp.float32),
            pltpu.VMEM((_K, D), jnp.float32),
            pltpu.VMEM((_K,), jnp.int32),
            pltpu.VMEM((_K,), jnp.int32),
            pltpu.VMEM_SHARED((N, D), jnp.float32),
            pltpu.SemaphoreType.DMA,
        ],
    )
    def k(h_hbm, eemb_hbm, src_hbm, dst_hbm, zeros_hbm, out_hbm,
          gbuf, ebuf, isrc, idst, acc, sem):
        c = lax.axis_index("c")
        s = lax.axis_index("s")
        wid = s * _NC + c
        _rows_copy(zeros_hbm, acc, s)
        plsc.subcore_barrier()
        e0 = wid * _EPW

        def body(i, carry):
            e = e0 + i * _K
            pltpu.sync_copy(src_hbm.at[pl.ds(e, _K)], isrc)
            pltpu.sync_copy(dst_hbm.at[pl.ds(e, _K)], idst)
            pltpu.sync_copy(eemb_hbm.at[pl.ds(e, _K)], ebuf)
            pltpu.async_copy(h_hbm.at[isrc], gbuf, sem).wait()

            def row(r, rc):
                for c8 in range(D // 16):
                    sl = pl.ds(c8 * 16, 16)
                    ebuf[r, sl] = jnp.maximum(ebuf[r, sl] + gbuf[r, sl], 0.0)
                return rc

            lax.fori_loop(0, _K, row, 0)
            pltpu.sync_copy(ebuf, acc.at[idst], add=True)
            return carry

        lax.fori_loop(0, _NBLK, body, 0)
        plsc.subcore_barrier()
        _rows_copy(acc, out_hbm.at[c], s)

    return k(h, eemb, src, dst, zeros_nd)


def _bn_tc(t, g, b):
    m = jnp.mean(t, axis=0, keepdims=True)
    d = t - m
    v = jnp.mean(d * d, axis=0, keepdims=True)
    return d * jax.lax.rsqrt(v + 1e-5) * g + b


def _mlp_body(final_relu, agg_ref, h_ref, sc_ref, w1_ref, b1_ref, g1_ref,
              t1_ref, w2_ref, b2_ref, g2_ref, t2_ref, out_ref):
    pre = (agg_ref[0] + agg_ref[1]) + sc_ref[0, 0] * h_ref[...]
    t = jnp.dot(pre, w1_ref[...], preferred_element_type=jnp.float32)
    t = jnp.maximum(_bn_tc(t + b1_ref[...], g1_ref[...], t1_ref[...]), 0.0)
    t = jnp.dot(t, w2_ref[...], preferred_element_type=jnp.float32)
    t = _bn_tc(t + b2_ref[...], g2_ref[...], t2_ref[...])
    if final_relu:
        t = jnp.maximum(t, 0.0)
    out_ref[...] = t


def _mlp(agg, h, scale, w1, b1, g1, t1, w2, b2, g2, t2, final_relu):
    nsmem = pl.BlockSpec(memory_space=pltpu.SMEM)
    args = (agg, h, scale, w1, b1, g1, t1, w2, b2, g2, t2)
    in_specs = [nsmem if a is scale else pl.BlockSpec(a.shape, None)
                for a in args]
    return pl.pallas_call(
        functools.partial(_mlp_body, final_relu),
        in_specs=in_specs,
        out_shape=jax.ShapeDtypeStruct((N, D), jnp.float32),
    )(*args)


def _mlp_pool_body(agg_ref, h_ref, sc_ref, w1_ref, b1_ref, g1_ref, t1_ref,
                   w2_ref, b2_ref, g2_ref, t2_ref, batch_ref, wp1_ref,
                   bp1_ref, wp_ref, bp_ref, out_ref, g_ref, hn_ref):
    pre = (agg_ref[0] + agg_ref[1]) + sc_ref[0, 0] * h_ref[...]
    t = jnp.dot(pre, w1_ref[...], preferred_element_type=jnp.float32)
    t = jnp.maximum(_bn_tc(t + b1_ref[...], g1_ref[...], t1_ref[...]), 0.0)
    t = jnp.dot(t, w2_ref[...], preferred_element_type=jnp.float32)
    hn = _bn_tc(t + b2_ref[...], g2_ref[...], t2_ref[...])
    hn_ref[...] = hn
    iota = lax.broadcasted_iota(jnp.int32, (N, B), 1)
    oh = (batch_ref[...] == iota).astype(jnp.float32)
    dn = (((0,), (0,)), ((), ()))
    sums = lax.dot_general(oh, hn, dn, preferred_element_type=jnp.float32)
    cnt = lax.dot_general(oh, jnp.ones((N, 1), jnp.float32), dn,
                          preferred_element_type=jnp.float32)
    hg = sums / jnp.maximum(cnt, 1.0)
    gg = jnp.maximum(hg, 0.0)
    gg = jnp.dot(gg, wp1_ref[...], preferred_element_type=jnp.float32)
    gg = jnp.maximum(gg + bp1_ref[...], 0.0)
    g_ref[...] = gg
    out = jnp.dot(gg, wp_ref[...], preferred_element_type=jnp.float32)
    out_ref[...] = out + bp_ref[...]


def _mlp_pool(agg, h, scale, w1, b1, g1, t1, w2, b2, g2, t2,
              batch2d, wp1, bp1, wp, bp):
    nsmem = pl.BlockSpec(memory_space=pltpu.SMEM)
    args = (agg, h, scale, w1, b1, g1, t1, w2, b2, g2, t2,
            batch2d, wp1, bp1, wp, bp)
    in_specs = [nsmem if a is scale else pl.BlockSpec(a.shape, None)
                for a in args]
    return pl.pallas_call(
        _mlp_pool_body,
        in_specs=in_specs,
        out_shape=[
            jax.ShapeDtypeStruct((B, C), jnp.float32),
            jax.ShapeDtypeStruct((B, D), jnp.float32),
            jax.ShapeDtypeStruct((N, D), jnp.float32),
        ],
    )(*args)


def kernel(x, edge_index, edge_attr, batch, node_table, We, be, eps,
           W1, b1, g1, bt1, W2, b2, g2, bt2, Wp1, bp1, Wp, bp):
    f32 = jnp.float32
    src = edge_index[0].astype(jnp.int32)
    dst = edge_index[1].astype(jnp.int32)
    ea8 = jnp.concatenate([edge_attr, jnp.zeros((E, 1), f32)], axis=1)
    w0 = jnp.concatenate([We[0], jnp.zeros((1, D), f32)], axis=0)
    w1e = jnp.concatenate([We[1], jnp.zeros((1, D), f32)], axis=0)
    c0 = node_table + be[0][None, :]
    c1 = be[1][None, :]
    zeros_nd = jnp.zeros((N, D), f32)
    batch2d = batch.astype(jnp.int32)[:, None]

    msg0, eemb1 = _edge_embed(ea8, w0, c0, w1e, c1)
    agg0 = _sc_scatter_add(msg0, dst, zeros_nd)
    sc0 = (1.0 + eps[0]).astype(f32).reshape(1, 1)
    h1 = _mlp(agg0, node_table, sc0, W1[0], b1[0][None], g1[0][None],
              bt1[0][None], W2[0], b2[0][None], g2[0][None], bt2[0][None],
              final_relu=True)
    agg1 = _sc_gather_msg_scatter(h1, eemb1, src, dst, zeros_nd)
    sc1 = (1.0 + eps[1]).astype(f32).reshape(1, 1)
    out, g, h_node = _mlp_pool(
        agg1, h1, sc1, W1[1], b1[1][None], g1[1][None], bt1[1][None],
        W2[1], b2[1][None], g2[1][None], bt2[1][None],
        batch2d, Wp1, bp1[None], Wp, bp[None])
    return out, g, h_node


# speedup vs baseline: 5.1002x; 1.6400x over previous
import functools

import jax
import jax.numpy as jnp
from jax import lax
from jax.experimental import pallas as pl
from jax.experimental.pallas import tpu as pltpu
from jax.experimental.pallas import tpu_sc as plsc

N = 10000
E = 320000
D = 128
B = 64
C = 10

_NC = 2
_NS = 16
_NW = _NC * _NS
_EPW = E // _NW
_KS = 80
_RS = 4
_NBLK_S = _EPW // _KS
_NOUT_S = (_NBLK_S + _RS - 1) // _RS
_KG = 40
_RG = 4
_NBLK_G = _EPW // _KG
_NOUT_G = (_NBLK_G + _RG - 1) // _RG
_RPT = 624
_TAIL0 = _RPT * _NS
_TAIL = N - _TAIL0


def _rows_copy(src_ref, dst_ref, s):
    r0 = s * _RPT
    pltpu.sync_copy(src_ref.at[pl.ds(r0, _RPT)], dst_ref.at[pl.ds(r0, _RPT)])

    @pl.when(s == _NS - 1)
    def _():
        pltpu.sync_copy(src_ref.at[pl.ds(_TAIL0, _TAIL)],
                        dst_ref.at[pl.ds(_TAIL0, _TAIL)])


_EBLK = 4000


def _edge_body(relu, ea_ref, w_ref, c_ref, out_ref):
    a = ea_ref[...].astype(jnp.bfloat16)
    w = w_ref[...].astype(jnp.bfloat16)
    t = jnp.dot(a, w, preferred_element_type=jnp.float32) + c_ref[...]
    out_ref[...] = jnp.maximum(t, 0.0) if relu else t


def _edge_embed(ea8, w, c, relu):
    return pl.pallas_call(
        functools.partial(_edge_body, relu),
        grid=(E // _EBLK,),
        in_specs=[
            pl.BlockSpec((_EBLK, 8), lambda i: (i, 0)),
            pl.BlockSpec((8, D), lambda i: (0, 0)),
            pl.BlockSpec((1, D), lambda i: (0, 0)),
        ],
        out_specs=pl.BlockSpec((_EBLK, D), lambda i: (i, 0)),
        out_shape=jax.ShapeDtypeStruct((E, D), jnp.float32),
    )(ea8, w, c)


def _sc_scatter_add(vals, idx, zeros_nd):
    mesh = plsc.VectorSubcoreMesh(core_axis_name="c", subcore_axis_name="s")
    scratch = ([pltpu.VMEM((_KS, D), jnp.float32)] * _RS +
               [pltpu.VMEM((_KS,), jnp.int32)] * _RS +
               [pltpu.VMEM_SHARED((N, D), jnp.float32)] +
               [pltpu.SemaphoreType.DMA] * (2 * _RS))

    @functools.partial(
        pl.kernel,
        mesh=mesh,
        out_type=jax.ShapeDtypeStruct((_NC, N, D), jnp.float32),
        scratch_types=scratch,
    )
    def k(vals_hbm, idx_hbm, zeros_hbm, out_hbm, *sc):
        vb = sc[0:_RS]
        ib = sc[_RS:2 * _RS]
        acc = sc[2 * _RS]
        sv = sc[2 * _RS + 1:3 * _RS + 1]
        si = sc[3 * _RS + 1:4 * _RS + 1]
        c = lax.axis_index("c")
        s = lax.axis_index("s")
        wid = s * _NC + c
        _rows_copy(zeros_hbm, acc, s)
        plsc.subcore_barrier()
        e0 = wid * _EPW
        for b in range(_RS):
            e = e0 + b * _KS
            pltpu.async_copy(vals_hbm.at[pl.ds(e, _KS)], vb[b], sv[b])
            pltpu.async_copy(idx_hbm.at[pl.ds(e, _KS)], ib[b], si[b])

        def outer(o, carry):
            for b in range(_RS):
                i = o * _RS + b

                @pl.when(i < _NBLK_S)
                def _(b=b):
                    pltpu.make_async_copy(
                        vals_hbm.at[pl.ds(0, _KS)], vb[b], sv[b]).wait()
                    pltpu.make_async_copy(
                        idx_hbm.at[pl.ds(0, _KS)], ib[b], si[b]).wait()
                    pltpu.sync_copy(vb[b], acc.at[ib[b]], add=True)

                @pl.when(i + _RS < _NBLK_S)
                def _(b=b, i=i):
                    e = e0 + (i + _RS) * _KS
                    pltpu.async_copy(vals_hbm.at[pl.ds(e, _KS)], vb[b], sv[b])
                    pltpu.async_copy(idx_hbm.at[pl.ds(e, _KS)], ib[b], si[b])
            return carry

        lax.fori_loop(0, _NOUT_S, outer, 0)
        plsc.subcore_barrier()
        _rows_copy(acc, out_hbm.at[c], s)

    return k(vals, idx, zeros_nd)


def _sc_gather_msg_scatter(h, eemb, src, dst, zeros_nd):
    mesh = plsc.VectorSubcoreMesh(core_axis_name="c", subcore_axis_name="s")
    scratch = ([pltpu.VMEM((_KG, D), jnp.float32)] * (2 * _RG) +
               [pltpu.VMEM((_KG,), jnp.int32)] * (2 * _RG) +
               [pltpu.VMEM_SHARED((N, D), jnp.float32)] +
               [pltpu.SemaphoreType.DMA] * (4 * _RG))

    @functools.partial(
        pl.kernel,
        mesh=mesh,
        out_type=jax.ShapeDtypeStruct((_NC, N, D), jnp.float32),
        scratch_types=scratch,
    )
    def k(h_hbm, eemb_hbm, src_hbm, dst_hbm, zeros_hbm, out_hbm, *sc):
        eb = sc[0:_RG]
        gb = sc[_RG:2 * _RG]
        isb = sc[2 * _RG:3 * _RG]
        idb = sc[3 * _RG:4 * _RG]
        acc = sc[4 * _RG]
        p = 4 * _RG + 1
        se = sc[p:p + _RG]
        sg = sc[p + _RG:p + 2 * _RG]
        ss = sc[p + 2 * _RG:p + 3 * _RG]
        sd = sc[p + 3 * _RG:p + 4 * _RG]
        c = lax.axis_index("c")
        s = lax.axis_index("s")
        wid = s * _NC + c
        _rows_copy(zeros_hbm, acc, s)
        plsc.subcore_barrier()
        e0 = wid * _EPW

        def loads(b, e):
            pltpu.async_copy(src_hbm.at[pl.ds(e, _KG)], isb[b], ss[b])
            pltpu.async_copy(dst_hbm.at[pl.ds(e, _KG)], idb[b], sd[b])
            pltpu.async_copy(eemb_hbm.at[pl.ds(e, _KG)], eb[b], se[b])

        for b in range(_RG):
            loads(b, e0 + b * _KG)
        pltpu.make_async_copy(src_hbm.at[pl.ds(0, _KG)], isb[0], ss[0]).wait()
        pltpu.async_copy(h_hbm.at[isb[0]], gb[0], sg[0])

        def outer(o, carry):
            for b in range(_RG):
                i = o * _RG + b
                bn = (b + 1) % _RG

                @pl.when(i + 1 < _NBLK_G)
                def _(bn=bn):
                    pltpu.make_async_copy(
                        src_hbm.at[pl.ds(0, _KG)], isb[bn], ss[bn]).wait()
                    pltpu.async_copy(h_hbm.at[isb[bn]], gb[bn], sg[bn])

                @pl.when(i < _NBLK_G)
                def _(b=b):
                    pltpu.make_async_copy(
                        eemb_hbm.at[pl.ds(0, _KG)], eb[b], se[b]).wait()
                    pltpu.make_async_copy(
                        h_hbm.at[isb[b]], gb[b], sg[b]).wait()

                    def row(r, rc, b=b):
                        for c8 in range(D // 16):
                            sl = pl.ds(c8 * 16, 16)
                            eb[b][r, sl] = jnp.maximum(
                                eb[b][r, sl] + gb[b][r, sl], 0.0)
                        return rc

                    lax.fori_loop(0, _KG, row, 0)
                    pltpu.make_async_copy(
                        dst_hbm.at[pl.ds(0, _KG)], idb[b], sd[b]).wait()
                    pltpu.sync_copy(eb[b], acc.at[idb[b]], add=True)

                @pl.when(i + _RG < _NBLK_G)
                def _(b=b, i=i):
                    loads(b, e0 + (i + _RG) * _KG)
            return carry

        lax.fori_loop(0, _NOUT_G, outer, 0)
        plsc.subcore_barrier()
        _rows_copy(acc, out_hbm.at[c], s)

    return k(h, eemb, src, dst, zeros_nd)


def _bn_tc(t, g, b):
    m = jnp.mean(t, axis=0, keepdims=True)
    d = t - m
    v = jnp.mean(d * d, axis=0, keepdims=True)
    return d * jax.lax.rsqrt(v + 1e-5) * g + b


def _mlp_body(final_relu, agg_ref, h_ref, sc_ref, w1_ref, b1_ref, g1_ref,
              t1_ref, w2_ref, b2_ref, g2_ref, t2_ref, out_ref):
    pre = (agg_ref[0] + agg_ref[1]) + sc_ref[0, 0] * h_ref[...]
    t = jnp.dot(pre, w1_ref[...], preferred_element_type=jnp.float32)
    t = jnp.maximum(_bn_tc(t + b1_ref[...], g1_ref[...], t1_ref[...]), 0.0)
    t = jnp.dot(t, w2_ref[...], preferred_element_type=jnp.float32)
    t = _bn_tc(t + b2_ref[...], g2_ref[...], t2_ref[...])
    if final_relu:
        t = jnp.maximum(t, 0.0)
    out_ref[...] = t


def _mlp(agg, h, scale, w1, b1, g1, t1, w2, b2, g2, t2, final_relu):
    nsmem = pl.BlockSpec(memory_space=pltpu.SMEM)
    args = (agg, h, scale, w1, b1, g1, t1, w2, b2, g2, t2)
    in_specs = [nsmem if a is scale else pl.BlockSpec(a.shape, None)
                for a in args]
    return pl.pallas_call(
        functools.partial(_mlp_body, final_relu),
        in_specs=in_specs,
        out_shape=jax.ShapeDtypeStruct((N, D), jnp.float32),
    )(*args)


def _mlp_pool_body(agg_ref, h_ref, sc_ref, w1_ref, b1_ref, g1_ref, t1_ref,
                   w2_ref, b2_ref, g2_ref, t2_ref, batch_ref, wp1_ref,
                   bp1_ref, wp_ref, bp_ref, out_ref, g_ref, hn_ref):
    pre = (agg_ref[0] + agg_ref[1]) + sc_ref[0, 0] * h_ref[...]
    t = jnp.dot(pre, w1_ref[...], preferred_element_type=jnp.float32)
    t = jnp.maximum(_bn_tc(t + b1_ref[...], g1_ref[...], t1_ref[...]), 0.0)
    t = jnp.dot(t, w2_ref[...], preferred_element_type=jnp.float32)
    hn = _bn_tc(t + b2_ref[...], g2_ref[...], t2_ref[...])
    hn_ref[...] = hn
    iota = lax.broadcasted_iota(jnp.int32, (N, B), 1)
    oh = (batch_ref[...] == iota).astype(jnp.float32)
    dn = (((0,), (0,)), ((), ()))
    sums = lax.dot_general(oh, hn, dn, preferred_element_type=jnp.float32)
    cnt = lax.dot_general(oh, jnp.ones((N, 1), jnp.float32), dn,
                          preferred_element_type=jnp.float32)
    hg = sums / jnp.maximum(cnt, 1.0)
    gg = jnp.maximum(hg, 0.0)
    gg = jnp.dot(gg, wp1_ref[...], preferred_element_type=jnp.float32)
    gg = jnp.maximum(gg + bp1_ref[...], 0.0)
    g_ref[...] = gg
    out = jnp.dot(gg, wp_ref[...], preferred_element_type=jnp.float32)
    out_ref[...] = out + bp_ref[...]


def _mlp_pool(agg, h, scale, w1, b1, g1, t1, w2, b2, g2, t2,
              batch2d, wp1, bp1, wp, bp):
    nsmem = pl.BlockSpec(memory_space=pltpu.SMEM)
    args = (agg, h, scale, w1, b1, g1, t1, w2, b2, g2, t2,
            batch2d, wp1, bp1, wp, bp)
    in_specs = [nsmem if a is scale else pl.BlockSpec(a.shape, None)
                for a in args]
    return pl.pallas_call(
        _mlp_pool_body,
        in_specs=in_specs,
        out_shape=[
            jax.ShapeDtypeStruct((B, C), jnp.float32),
            jax.ShapeDtypeStruct((B, D), jnp.float32),
            jax.ShapeDtypeStruct((N, D), jnp.float32),
        ],
    )(*args)


def kernel(x, edge_index, edge_attr, batch, node_table, We, be, eps,
           W1, b1, g1, bt1, W2, b2, g2, bt2, Wp1, bp1, Wp, bp):
    f32 = jnp.float32
    src = edge_index[0].astype(jnp.int32)
    dst = edge_index[1].astype(jnp.int32)
    ea8 = jnp.concatenate([edge_attr, jnp.zeros((E, 1), f32)], axis=1)
    w0 = jnp.concatenate([We[0], jnp.zeros((1, D), f32)], axis=0)
    w1e = jnp.concatenate([We[1], jnp.zeros((1, D), f32)], axis=0)
    c0 = node_table + be[0][None, :]
    c1 = be[1][None, :]
    zeros_nd = jnp.zeros((N, D), f32)
    batch2d = batch.astype(jnp.int32)[:, None]

    msg0 = _edge_embed(ea8, w0, c0, relu=True)
    agg0 = _sc_scatter_add(msg0, dst, zeros_nd)
    eemb1 = _edge_embed(ea8, w1e, c1, relu=False)
    sc0 = (1.0 + eps[0]).astype(f32).reshape(1, 1)
    h1 = _mlp(agg0, node_table, sc0, W1[0], b1[0][None], g1[0][None],
              bt1[0][None], W2[0], b2[0][None], g2[0][None], bt2[0][None],
              final_relu=True)
    agg1 = _sc_gather_msg_scatter(h1, eemb1, src, dst, zeros_nd)
    sc1 = (1.0 + eps[1]).astype(f32).reshape(1, 1)
    out, g, h_node = _mlp_pool(
        agg1, h1, sc1, W1[1], b1[1][None], g1[1][None], bt1[1][None],
        W2[1], b2[1][None], g2[1][None], bt2[1][None],
        batch2d, Wp1, bp1[None], Wp, bp[None])
    return out, g, h_node


# speedup vs baseline: 5.1474x; 1.0092x over previous
import functools

import jax
import jax.numpy as jnp
from jax import lax
from jax.experimental import pallas as pl
from jax.experimental.pallas import tpu as pltpu
from jax.experimental.pallas import tpu_sc as plsc

N = 10000
E = 320000
D = 128
B = 64
C = 10

_NC = 2
_NS = 16
_NW = _NC * _NS
_EPW = E // _NW
_KS = 80
_RS = 4
_NBLK_S = _EPW // _KS
_NOUT_S = (_NBLK_S + _RS - 1) // _RS
_KG = 40
_RG = 4
_NBLK_G = _EPW // _KG
_NOUT_G = (_NBLK_G + _RG - 1) // _RG
_RPT = 624
_TAIL0 = _RPT * _NS
_TAIL = N - _TAIL0


def _rows_copy(src_ref, dst_ref, s):
    r0 = s * _RPT
    pltpu.sync_copy(src_ref.at[pl.ds(r0, _RPT)], dst_ref.at[pl.ds(r0, _RPT)])

    @pl.when(s == _NS - 1)
    def _():
        pltpu.sync_copy(src_ref.at[pl.ds(_TAIL0, _TAIL)],
                        dst_ref.at[pl.ds(_TAIL0, _TAIL)])


_ZR = 48


def _zero_acc(zb, acc, s):
    def zrow(r, carry):
        for c8 in range(D // 16):
            zb[r, pl.ds(c8 * 16, 16)] = jnp.zeros((16,), jnp.float32)
        return carry

    lax.fori_loop(0, _ZR, zrow, 0)
    r0 = s * _RPT
    for j in range(_RPT // _ZR):
        pltpu.sync_copy(zb, acc.at[pl.ds(r0 + j * _ZR, _ZR)])

    @pl.when(s == _NS - 1)
    def _():
        pltpu.sync_copy(zb.at[pl.ds(0, _TAIL)], acc.at[pl.ds(_TAIL0, _TAIL)])


_EBLK = 4000


def _edge_body(relu, ea_ref, w_ref, c_ref, out_ref):
    a = ea_ref[...].astype(jnp.bfloat16)
    w = w_ref[...].astype(jnp.bfloat16)
    t = jnp.dot(a, w, preferred_element_type=jnp.float32) + c_ref[...]
    out_ref[...] = jnp.maximum(t, 0.0) if relu else t


def _edge_embed(ea8, w, c, relu):
    return pl.pallas_call(
        functools.partial(_edge_body, relu),
        grid=(E // _EBLK,),
        in_specs=[
            pl.BlockSpec((_EBLK, 8), lambda i: (i, 0)),
            pl.BlockSpec((8, D), lambda i: (0, 0)),
            pl.BlockSpec((1, D), lambda i: (0, 0)),
        ],
        out_specs=pl.BlockSpec((_EBLK, D), lambda i: (i, 0)),
        out_shape=jax.ShapeDtypeStruct((E, D), jnp.float32),
    )(ea8, w, c)


def _sc_scatter_add(vals, dst):
    mesh = plsc.VectorSubcoreMesh(core_axis_name="c", subcore_axis_name="s")
    scratch = ([pltpu.VMEM((_KS, D), jnp.float32)] * _RS +
               [pltpu.VMEM((_KS,), jnp.int32)] * _RS +
               [pltpu.VMEM((_ZR, D), jnp.float32)] +
               [pltpu.VMEM_SHARED((N, D), jnp.float32)] +
               [pltpu.SemaphoreType.DMA] * (2 * _RS))

    @functools.partial(
        pl.kernel,
        mesh=mesh,
        out_type=jax.ShapeDtypeStruct((_NC, N, D), jnp.float32),
        scratch_types=scratch,
    )
    def k(vals_hbm, dst_hbm, out_hbm, *sc):
        vb = sc[0:_RS]
        ib = sc[_RS:2 * _RS]
        zb = sc[2 * _RS]
        acc = sc[2 * _RS + 1]
        sv = sc[2 * _RS + 2:3 * _RS + 2]
        si = sc[3 * _RS + 2:4 * _RS + 2]
        c = lax.axis_index("c")
        s = lax.axis_index("s")
        wid = s * _NC + c
        _zero_acc(zb, acc, s)
        plsc.subcore_barrier()
        e0 = wid * _EPW
        for b in range(_RS):
            e = e0 + b * _KS
            pltpu.async_copy(vals_hbm.at[pl.ds(e, _KS)], vb[b], sv[b])
            pltpu.async_copy(dst_hbm.at[pl.ds(e, _KS)], ib[b], si[b])

        def outer(o, carry):
            for b in range(_RS):
                i = o * _RS + b

                @pl.when(i < _NBLK_S)
                def _(b=b):
                    pltpu.make_async_copy(
                        vals_hbm.at[pl.ds(0, _KS)], vb[b], sv[b]).wait()
                    pltpu.make_async_copy(
                        dst_hbm.at[pl.ds(0, _KS)], ib[b], si[b]).wait()
                    pltpu.sync_copy(vb[b], acc.at[ib[b]], add=True)

                @pl.when(i + _RS < _NBLK_S)
                def _(b=b, i=i):
                    e = e0 + (i + _RS) * _KS
                    pltpu.async_copy(vals_hbm.at[pl.ds(e, _KS)], vb[b], sv[b])
                    pltpu.async_copy(dst_hbm.at[pl.ds(e, _KS)], ib[b], si[b])
            return carry

        lax.fori_loop(0, _NOUT_S, outer, 0)
        plsc.subcore_barrier()
        _rows_copy(acc, out_hbm.at[c], s)

    return k(vals, dst)


def _sc_gather_msg_scatter(h, eemb, src, dst):
    mesh = plsc.VectorSubcoreMesh(core_axis_name="c", subcore_axis_name="s")
    scratch = ([pltpu.VMEM((_KG, D), jnp.float32)] * (2 * _RG) +
               [pltpu.VMEM((_KG,), jnp.int32)] * (2 * _RG) +
               [pltpu.VMEM((_ZR, D), jnp.float32)] +
               [pltpu.VMEM_SHARED((N, D), jnp.float32)] +
               [pltpu.SemaphoreType.DMA] * (4 * _RG))

    @functools.partial(
        pl.kernel,
        mesh=mesh,
        out_type=jax.ShapeDtypeStruct((_NC, N, D), jnp.float32),
        scratch_types=scratch,
    )
    def k(h_hbm, eemb_hbm, src_hbm, dst_hbm, out_hbm, *sc):
        eb = sc[0:_RG]
        gb = sc[_RG:2 * _RG]
        isb = sc[2 * _RG:3 * _RG]
        idb = sc[3 * _RG:4 * _RG]
        zb = sc[4 * _RG]
        acc = sc[4 * _RG + 1]
        p = 4 * _RG + 2
        se = sc[p:p + _RG]
        sg = sc[p + _RG:p + 2 * _RG]
        ss = sc[p + 2 * _RG:p + 3 * _RG]
        sd = sc[p + 3 * _RG:p + 4 * _RG]
        c = lax.axis_index("c")
        s = lax.axis_index("s")
        wid = s * _NC + c
        _zero_acc(zb, acc, s)
        plsc.subcore_barrier()
        e0 = wid * _EPW

        def loads(b, e):
            pltpu.async_copy(src_hbm.at[pl.ds(e, _KG)], isb[b], ss[b])
            pltpu.async_copy(dst_hbm.at[pl.ds(e, _KG)], idb[b], sd[b])
            pltpu.async_copy(eemb_hbm.at[pl.ds(e, _KG)], eb[b], se[b])

        for b in range(_RG):
            loads(b, e0 + b * _KG)
        pltpu.make_async_copy(src_hbm.at[pl.ds(0, _KG)], isb[0], ss[0]).wait()
        pltpu.async_copy(h_hbm.at[isb[0]], gb[0], sg[0])

        def outer(o, carry):
            for b in range(_RG):
                i = o * _RG + b
                bn = (b + 1) % _RG

                @pl.when(i + 1 < _NBLK_G)
                def _(bn=bn):
                    pltpu.make_async_copy(
                        src_hbm.at[pl.ds(0, _KG)], isb[bn], ss[bn]).wait()
                    pltpu.async_copy(h_hbm.at[isb[bn]], gb[bn], sg[bn])

                @pl.when(i < _NBLK_G)
                def _(b=b):
                    pltpu.make_async_copy(
                        eemb_hbm.at[pl.ds(0, _KG)], eb[b], se[b]).wait()
                    pltpu.make_async_copy(
                        h_hbm.at[isb[b]], gb[b], sg[b]).wait()

                    def row(r, rc, b=b):
                        for c8 in range(D // 16):
                            sl = pl.ds(c8 * 16, 16)
                            eb[b][r, sl] = jnp.maximum(
                                eb[b][r, sl] + gb[b][r, sl], 0.0)
                        return rc

                    lax.fori_loop(0, _KG, row, 0)
                    pltpu.make_async_copy(
                        dst_hbm.at[pl.ds(0, _KG)], idb[b], sd[b]).wait()
                    pltpu.sync_copy(eb[b], acc.at[idb[b]], add=True)

                @pl.when(i + _RG < _NBLK_G)
                def _(b=b, i=i):
                    loads(b, e0 + (i + _RG) * _KG)
            return carry

        lax.fori_loop(0, _NOUT_G, outer, 0)
        plsc.subcore_barrier()
        _rows_copy(acc, out_hbm.at[c], s)

    return k(h, eemb, src, dst)


def _bn_tc(t, g, b):
    m = jnp.mean(t, axis=0, keepdims=True)
    d = t - m
    v = jnp.mean(d * d, axis=0, keepdims=True)
    return d * jax.lax.rsqrt(v + 1e-5) * g + b


def _mlp_body(final_relu, agg_ref, h_ref, sc_ref, w1_ref, b1_ref, g1_ref,
              t1_ref, w2_ref, b2_ref, g2_ref, t2_ref, out_ref):
    pre = (agg_ref[0] + agg_ref[1]) + sc_ref[0, 0] * h_ref[...]
    t = jnp.dot(pre, w1_ref[...], preferred_element_type=jnp.float32)
    t = jnp.maximum(_bn_tc(t + b1_ref[...], g1_ref[...], t1_ref[...]), 0.0)
    t = jnp.dot(t, w2_ref[...], preferred_element_type=jnp.float32)
    t = _bn_tc(t + b2_ref[...], g2_ref[...], t2_ref[...])
    if final_relu:
        t = jnp.maximum(t, 0.0)
    out_ref[...] = t


def _mlp(agg, h, scale, w1, b1, g1, t1, w2, b2, g2, t2, final_relu):
    nsmem = pl.BlockSpec(memory_space=pltpu.SMEM)
    args = (agg, h, scale, w1, b1, g1, t1, w2, b2, g2, t2)
    in_specs = [nsmem if a is scale else pl.BlockSpec(a.shape, None)
                for a in args]
    return pl.pallas_call(
        functools.partial(_mlp_body, final_relu),
        in_specs=in_specs,
        out_shape=jax.ShapeDtypeStruct((N, D), jnp.float32),
    )(*args)


def _mlp_pool_body(agg_ref, h_ref, sc_ref, w1_ref, b1_ref, g1_ref, t1_ref,
                   w2_ref, b2_ref, g2_ref, t2_ref, batch_ref, wp1_ref,
                   bp1_ref, wp_ref, bp_ref, out_ref, g_ref, hn_ref):
    pre = (agg_ref[0] + agg_ref[1]) + sc_ref[0, 0] * h_ref[...]
    t = jnp.dot(pre, w1_ref[...], preferred_element_type=jnp.float32)
    t = jnp.maximum(_bn_tc(t + b1_ref[...], g1_ref[...], t1_ref[...]), 0.0)
    t = jnp.dot(t, w2_ref[...], preferred_element_type=jnp.float32)
    hn = _bn_tc(t + b2_ref[...], g2_ref[...], t2_ref[...])
    hn_ref[...] = hn
    iota = lax.broadcasted_iota(jnp.int32, (N, B), 1)
    oh = (batch_ref[...] == iota).astype(jnp.float32)
    dn = (((0,), (0,)), ((), ()))
    sums = lax.dot_general(oh, hn, dn, preferred_element_type=jnp.float32)
    cnt = lax.dot_general(oh, jnp.ones((N, 1), jnp.float32), dn,
                          preferred_element_type=jnp.float32)
    hg = sums / jnp.maximum(cnt, 1.0)
    gg = jnp.maximum(hg, 0.0)
    gg = jnp.dot(gg, wp1_ref[...], preferred_element_type=jnp.float32)
    gg = jnp.maximum(gg + bp1_ref[...], 0.0)
    g_ref[...] = gg
    out = jnp.dot(gg, wp_ref[...], preferred_element_type=jnp.float32)
    out_ref[...] = out + bp_ref[...]


def _mlp_pool(agg, h, scale, w1, b1, g1, t1, w2, b2, g2, t2,
              batch2d, wp1, bp1, wp, bp):
    nsmem = pl.BlockSpec(memory_space=pltpu.SMEM)
    args = (agg, h, scale, w1, b1, g1, t1, w2, b2, g2, t2,
            batch2d, wp1, bp1, wp, bp)
    in_specs = [nsmem if a is scale else pl.BlockSpec(a.shape, None)
                for a in args]
    return pl.pallas_call(
        _mlp_pool_body,
        in_specs=in_specs,
        out_shape=[
            jax.ShapeDtypeStruct((B, C), jnp.float32),
            jax.ShapeDtypeStruct((B, D), jnp.float32),
            jax.ShapeDtypeStruct((N, D), jnp.float32),
        ],
    )(*args)


def kernel(x, edge_index, edge_attr, batch, node_table, We, be, eps,
           W1, b1, g1, bt1, W2, b2, g2, bt2, Wp1, bp1, Wp, bp):
    f32 = jnp.float32
    src = edge_index[0].astype(jnp.int32)
    dst = edge_index[1].astype(jnp.int32)
    ea8 = jnp.concatenate([edge_attr, jnp.zeros((E, 1), f32)], axis=1)
    w0 = jnp.concatenate([We[0], jnp.zeros((1, D), f32)], axis=0)
    w1e = jnp.concatenate([We[1], jnp.zeros((1, D), f32)], axis=0)
    c0 = node_table + be[0][None, :]
    c1 = be[1][None, :]
    batch2d = batch.astype(jnp.int32)[:, None]

    msg0 = _edge_embed(ea8, w0, c0, relu=True)
    agg0 = _sc_scatter_add(msg0, dst)
    eemb1 = _edge_embed(ea8, w1e, c1, relu=False)
    sc0 = (1.0 + eps[0]).astype(f32).reshape(1, 1)
    h1 = _mlp(agg0, node_table, sc0, W1[0], b1[0][None], g1[0][None],
              bt1[0][None], W2[0], b2[0][None], g2[0][None], bt2[0][None],
              final_relu=True)
    agg1 = _sc_gather_msg_scatter(h1, eemb1, src, dst)
    sc1 = (1.0 + eps[1]).astype(f32).reshape(1, 1)
    out, g, h_node = _mlp_pool(
        agg1, h1, sc1, W1[1], b1[1][None], g1[1][None], bt1[1][None],
        W2[1], b2[1][None], g2[1][None], bt2[1][None],
        batch2d, Wp1, bp1[None], Wp, bp[None])
    return out, g, h_node


# speedup vs baseline: 5.3430x; 1.0380x over previous
import functools

import jax
import jax.numpy as jnp
from jax import lax
from jax.experimental import pallas as pl
from jax.experimental.pallas import tpu as pltpu
from jax.experimental.pallas import tpu_sc as plsc

N = 10000
E = 320000
D = 128
B = 64
C = 10

_NC = 2
_NS = 16
_NW = _NC * _NS
_EPW = E // _NW
_KS = 80
_RS = 4
_NBLK_S = _EPW // _KS
_KG = 40
_RG = 4
_NBLK_G = _EPW // _KG
_PF = 2
_NOUT_S = (_NBLK_S + _PF + _RS - 1) // _RS
_NOUT_G = (_NBLK_G + _PF + _RG - 1) // _RG
_RPT = 624
_TAIL0 = _RPT * _NS
_TAIL = N - _TAIL0


def _rows_copy(src_ref, dst_ref, s):
    r0 = s * _RPT
    pltpu.sync_copy(src_ref.at[pl.ds(r0, _RPT)], dst_ref.at[pl.ds(r0, _RPT)])

    @pl.when(s == _NS - 1)
    def _():
        pltpu.sync_copy(src_ref.at[pl.ds(_TAIL0, _TAIL)],
                        dst_ref.at[pl.ds(_TAIL0, _TAIL)])


_ZR = 48


def _zero_acc(zb, acc, s):
    def zrow(r, carry):
        for c8 in range(D // 16):
            zb[r, pl.ds(c8 * 16, 16)] = jnp.zeros((16,), jnp.float32)
        return carry

    lax.fori_loop(0, _ZR, zrow, 0)
    r0 = s * _RPT
    for j in range(_RPT // _ZR):
        pltpu.sync_copy(zb, acc.at[pl.ds(r0 + j * _ZR, _ZR)])

    @pl.when(s == _NS - 1)
    def _():
        pltpu.sync_copy(zb.at[pl.ds(0, _TAIL)], acc.at[pl.ds(_TAIL0, _TAIL)])


_EBLK = 4000


def _edge_body(relu, ea_ref, w_ref, c_ref, out_ref):
    a = ea_ref[...].astype(jnp.bfloat16)
    w = w_ref[...].astype(jnp.bfloat16)
    t = jnp.dot(a, w, preferred_element_type=jnp.float32) + c_ref[...]
    out_ref[...] = jnp.maximum(t, 0.0) if relu else t


def _edge_embed(ea8, w, c, relu):
    return pl.pallas_call(
        functools.partial(_edge_body, relu),
        grid=(E // _EBLK,),
        in_specs=[
            pl.BlockSpec((_EBLK, 8), lambda i: (i, 0)),
            pl.BlockSpec((8, D), lambda i: (0, 0)),
            pl.BlockSpec((1, D), lambda i: (0, 0)),
        ],
        out_specs=pl.BlockSpec((_EBLK, D), lambda i: (i, 0)),
        out_shape=jax.ShapeDtypeStruct((E, D), jnp.float32),
    )(ea8, w, c)


def _sc_scatter_add(vals, dst):
    mesh = plsc.VectorSubcoreMesh(core_axis_name="c", subcore_axis_name="s")
    scratch = ([pltpu.VMEM((_KS, D), jnp.float32)] * _RS +
               [pltpu.VMEM((_KS,), jnp.int32)] * _RS +
               [pltpu.VMEM((_ZR, D), jnp.float32)] +
               [pltpu.VMEM_SHARED((N, D), jnp.float32)] +
               [pltpu.SemaphoreType.DMA] * (3 * _RS))

    @functools.partial(
        pl.kernel,
        mesh=mesh,
        out_type=jax.ShapeDtypeStruct((_NC, N, D), jnp.float32),
        scratch_types=scratch,
    )
    def k(vals_hbm, dst_hbm, out_hbm, *sc):
        vb = sc[0:_RS]
        ib = sc[_RS:2 * _RS]
        zb = sc[2 * _RS]
        acc = sc[2 * _RS + 1]
        sv = sc[2 * _RS + 2:3 * _RS + 2]
        si = sc[3 * _RS + 2:4 * _RS + 2]
        sw = sc[4 * _RS + 2:5 * _RS + 2]
        c = lax.axis_index("c")
        s = lax.axis_index("s")
        wid = s * _NC + c
        _zero_acc(zb, acc, s)
        plsc.subcore_barrier()
        e0 = wid * _EPW

        def loads(b, e):
            pltpu.async_copy(vals_hbm.at[pl.ds(e, _KS)], vb[b], sv[b])
            pltpu.async_copy(dst_hbm.at[pl.ds(e, _KS)], ib[b], si[b])

        for b in range(_PF):
            loads(b, e0 + b * _KS)

        def outer(o, carry):
            for b in range(_RS):
                j = o * _RS + b
                bf = (b + _PF) % _RS

                @pl.when(jnp.logical_and(j >= _PF, j < _NBLK_S + _PF))
                def _(bf=bf):
                    pltpu.make_async_copy(vb[bf], acc.at[ib[bf]],
                                          sw[bf]).wait()

                @pl.when(j + _PF < _NBLK_S)
                def _(bf=bf, j=j):
                    loads(bf, e0 + (j + _PF) * _KS)

                @pl.when(j < _NBLK_S)
                def _(b=b):
                    pltpu.make_async_copy(
                        vals_hbm.at[pl.ds(0, _KS)], vb[b], sv[b]).wait()
                    pltpu.make_async_copy(
                        dst_hbm.at[pl.ds(0, _KS)], ib[b], si[b]).wait()
                    pltpu.async_copy(vb[b], acc.at[ib[b]], sw[b], add=True)
            return carry

        lax.fori_loop(0, _NOUT_S, outer, 0)
        plsc.subcore_barrier()
        _rows_copy(acc, out_hbm.at[c], s)

    return k(vals, dst)


def _sc_gather_msg_scatter(h, eemb, src, dst):
    mesh = plsc.VectorSubcoreMesh(core_axis_name="c", subcore_axis_name="s")
    scratch = ([pltpu.VMEM((_KG, D), jnp.float32)] * (2 * _RG) +
               [pltpu.VMEM((_KG,), jnp.int32)] * (2 * _RG) +
               [pltpu.VMEM((_ZR, D), jnp.float32)] +
               [pltpu.VMEM_SHARED((N, D), jnp.float32)] +
               [pltpu.SemaphoreType.DMA] * (5 * _RG))

    @functools.partial(
        pl.kernel,
        mesh=mesh,
        out_type=jax.ShapeDtypeStruct((_NC, N, D), jnp.float32),
        scratch_types=scratch,
    )
    def k(h_hbm, eemb_hbm, src_hbm, dst_hbm, out_hbm, *sc):
        eb = sc[0:_RG]
        gb = sc[_RG:2 * _RG]
        isb = sc[2 * _RG:3 * _RG]
        idb = sc[3 * _RG:4 * _RG]
        zb = sc[4 * _RG]
        acc = sc[4 * _RG + 1]
        p = 4 * _RG + 2
        se = sc[p:p + _RG]
        sg = sc[p + _RG:p + 2 * _RG]
        ss = sc[p + 2 * _RG:p + 3 * _RG]
        sd = sc[p + 3 * _RG:p + 4 * _RG]
        sw = sc[p + 4 * _RG:p + 5 * _RG]
        c = lax.axis_index("c")
        s = lax.axis_index("s")
        wid = s * _NC + c
        _zero_acc(zb, acc, s)
        plsc.subcore_barrier()
        e0 = wid * _EPW

        def loads(b, e):
            pltpu.async_copy(src_hbm.at[pl.ds(e, _KG)], isb[b], ss[b])
            pltpu.async_copy(dst_hbm.at[pl.ds(e, _KG)], idb[b], sd[b])
            pltpu.async_copy(eemb_hbm.at[pl.ds(e, _KG)], eb[b], se[b])

        for b in range(_PF):
            loads(b, e0 + b * _KG)
        pltpu.make_async_copy(src_hbm.at[pl.ds(0, _KG)], isb[0], ss[0]).wait()
        pltpu.async_copy(h_hbm.at[isb[0]], gb[0], sg[0])

        def outer(o, carry):
            for b in range(_RG):
                j = o * _RG + b
                bn = (b + 1) % _RG
                bf = (b + _PF) % _RG

                @pl.when(jnp.logical_and(j >= _PF, j < _NBLK_G + _PF))
                def _(bf=bf):
                    pltpu.make_async_copy(eb[bf], acc.at[idb[bf]],
                                          sw[bf]).wait()

                @pl.when(j + _PF < _NBLK_G)
                def _(bf=bf, j=j):
                    loads(bf, e0 + (j + _PF) * _KG)

                @pl.when(j + 1 < _NBLK_G)
                def _(bn=bn):
                    pltpu.make_async_copy(
                        src_hbm.at[pl.ds(0, _KG)], isb[bn], ss[bn]).wait()
                    pltpu.async_copy(h_hbm.at[isb[bn]], gb[bn], sg[bn])

                @pl.when(j < _NBLK_G)
                def _(b=b):
                    pltpu.make_async_copy(
                        eemb_hbm.at[pl.ds(0, _KG)], eb[b], se[b]).wait()
                    pltpu.make_async_copy(
                        h_hbm.at[isb[b]], gb[b], sg[b]).wait()

                    def row(r, rc, b=b):
                        for c8 in range(D // 16):
                            sl = pl.ds(c8 * 16, 16)
                            eb[b][r, sl] = jnp.maximum(
                                eb[b][r, sl] + gb[b][r, sl], 0.0)
                        return rc

                    lax.fori_loop(0, _KG, row, 0)
                    pltpu.make_async_copy(
                        dst_hbm.at[pl.ds(0, _KG)], idb[b], sd[b]).wait()
                    pltpu.async_copy(eb[b], acc.at[idb[b]], sw[b], add=True)
            return carry

        lax.fori_loop(0, _NOUT_G, outer, 0)
        plsc.subcore_barrier()
        _rows_copy(acc, out_hbm.at[c], s)

    return k(h, eemb, src, dst)


def _bn_tc(t, g, b):
    m = jnp.mean(t, axis=0, keepdims=True)
    d = t - m
    v = jnp.mean(d * d, axis=0, keepdims=True)
    return d * jax.lax.rsqrt(v + 1e-5) * g + b


def _mlp_body(final_relu, agg_ref, h_ref, sc_ref, w1_ref, b1_ref, g1_ref,
              t1_ref, w2_ref, b2_ref, g2_ref, t2_ref, out_ref):
    pre = (agg_ref[0] + agg_ref[1]) + sc_ref[0, 0] * h_ref[...]
    t = jnp.dot(pre, w1_ref[...], preferred_element_type=jnp.float32)
    t = jnp.maximum(_bn_tc(t + b1_ref[...], g1_ref[...], t1_ref[...]), 0.0)
    t = jnp.dot(t, w2_ref[...], preferred_element_type=jnp.float32)
    t = _bn_tc(t + b2_ref[...], g2_ref[...], t2_ref[...])
    if final_relu:
        t = jnp.maximum(t, 0.0)
    out_ref[...] = t


def _mlp(agg, h, scale, w1, b1, g1, t1, w2, b2, g2, t2, final_relu):
    nsmem = pl.BlockSpec(memory_space=pltpu.SMEM)
    args = (agg, h, scale, w1, b1, g1, t1, w2, b2, g2, t2)
    in_specs = [nsmem if a is scale else pl.BlockSpec(a.shape, None)
                for a in args]
    return pl.pallas_call(
        functools.partial(_mlp_body, final_relu),
        in_specs=in_specs,
        out_shape=jax.ShapeDtypeStruct((N, D), jnp.float32),
    )(*args)


def _mlp_pool_body(agg_ref, h_ref, sc_ref, w1_ref, b1_ref, g1_ref, t1_ref,
                   w2_ref, b2_ref, g2_ref, t2_ref, batch_ref, wp1_ref,
                   bp1_ref, wp_ref, bp_ref, out_ref, g_ref, hn_ref):
    pre = (agg_ref[0] + agg_ref[1]) + sc_ref[0, 0] * h_ref[...]
    t = jnp.dot(pre, w1_ref[...], preferred_element_type=jnp.float32)
    t = jnp.maximum(_bn_tc(t + b1_ref[...], g1_ref[...], t1_ref[...]), 0.0)
    t = jnp.dot(t, w2_ref[...], preferred_element_type=jnp.float32)
    hn = _bn_tc(t + b2_ref[...], g2_ref[...], t2_ref[...])
    hn_ref[...] = hn
    iota = lax.broadcasted_iota(jnp.int32, (N, B), 1)
    oh = (batch_ref[...] == iota).astype(jnp.float32)
    dn = (((0,), (0,)), ((), ()))
    sums = lax.dot_general(oh, hn, dn, preferred_element_type=jnp.float32)
    cnt = lax.dot_general(oh, jnp.ones((N, 1), jnp.float32), dn,
                          preferred_element_type=jnp.float32)
    hg = sums / jnp.maximum(cnt, 1.0)
    gg = jnp.maximum(hg, 0.0)
    gg = jnp.dot(gg, wp1_ref[...], preferred_element_type=jnp.float32)
    gg = jnp.maximum(gg + bp1_ref[...], 0.0)
    g_ref[...] = gg
    out = jnp.dot(gg, wp_ref[...], preferred_element_type=jnp.float32)
    out_ref[...] = out + bp_ref[...]


def _mlp_pool(agg, h, scale, w1, b1, g1, t1, w2, b2, g2, t2,
              batch2d, wp1, bp1, wp, bp):
    nsmem = pl.BlockSpec(memory_space=pltpu.SMEM)
    args = (agg, h, scale, w1, b1, g1, t1, w2, b2, g2, t2,
            batch2d, wp1, bp1, wp, bp)
    in_specs = [nsmem if a is scale else pl.BlockSpec(a.shape, None)
                for a in args]
    return pl.pallas_call(
        _mlp_pool_body,
        in_specs=in_specs,
        out_shape=[
            jax.ShapeDtypeStruct((B, C), jnp.float32),
            jax.ShapeDtypeStruct((B, D), jnp.float32),
            jax.ShapeDtypeStruct((N, D), jnp.float32),
        ],
    )(*args)


def kernel(x, edge_index, edge_attr, batch, node_table, We, be, eps,
           W1, b1, g1, bt1, W2, b2, g2, bt2, Wp1, bp1, Wp, bp):
    f32 = jnp.float32
    src = edge_index[0].astype(jnp.int32)
    dst = edge_index[1].astype(jnp.int32)
    ea8 = jnp.concatenate([edge_attr, jnp.zeros((E, 1), f32)], axis=1)
    w0 = jnp.concatenate([We[0], jnp.zeros((1, D), f32)], axis=0)
    w1e = jnp.concatenate([We[1], jnp.zeros((1, D), f32)], axis=0)
    c0 = node_table + be[0][None, :]
    c1 = be[1][None, :]
    batch2d = batch.astype(jnp.int32)[:, None]

    msg0 = _edge_embed(ea8, w0, c0, relu=True)
    agg0 = _sc_scatter_add(msg0, dst)
    eemb1 = _edge_embed(ea8, w1e, c1, relu=False)
    sc0 = (1.0 + eps[0]).astype(f32).reshape(1, 1)
    h1 = _mlp(agg0, node_table, sc0, W1[0], b1[0][None], g1[0][None],
              bt1[0][None], W2[0], b2[0][None], g2[0][None], bt2[0][None],
              final_relu=True)
    agg1 = _sc_gather_msg_scatter(h1, eemb1, src, dst)
    sc1 = (1.0 + eps[1]).astype(f32).reshape(1, 1)
    out, g, h_node = _mlp_pool(
        agg1, h1, sc1, W1[1], b1[1][None], g1[1][None], bt1[1][None],
        W2[1], b2[1][None], g2[1][None], bt2[1][None],
        batch2d, Wp1, bp1[None], Wp, bp[None])
    return out, g, h_node
